# baseline (speedup 1.0000x reference)
import functools

import jax
import jax.numpy as jnp
from jax import lax
from jax.experimental import pallas as pl
from jax.experimental.pallas import tpu as pltpu
from jax.experimental.pallas import tpu_sc as plsc

S = 64.0
M = 0.4
_MS = M * S

_B = 1024
_V = 100000
_NW = 32
_ROWS_PER_W = _B // _NW
_CHUNK = 20000
_CHUNKS_PER_ROW = _V // _CHUNK
_VECS = _CHUNK // 16


def _sc_body(logits_hbm, labels_hbm, out_hbm, buf, labels_v, sem):
    cid = lax.axis_index("c")
    sid = lax.axis_index("s")
    wid = sid * 2 + cid
    r0 = wid * _ROWS_PER_W
    base = r0 * _V

    pltpu.sync_copy(labels_hbm.at[pl.ds(r0, _ROWS_PER_W)],
                    labels_v.at[pl.ds(0, _ROWS_PER_W)])

    def chunk_step(t, carry):
        off = base + t * _CHUNK
        pltpu.sync_copy(logits_hbm.at[pl.ds(off, _CHUNK)], buf)

        def vec_step(i, c):
            buf[pl.ds(i * 16, 16)] = buf[pl.ds(i * 16, 16)] * S
            return c

        lax.fori_loop(0, _VECS, vec_step, 0, unroll=8)

        r = t // _CHUNKS_PER_ROW
        c0 = (t - r * _CHUNKS_PER_ROW) * _CHUNK
        lab = labels_v[pl.ds(r, 16)][0]
        col = lab - c0

        @pl.when(jnp.logical_and(col >= 0, col < _CHUNK))
        def _fix():
            vbase = (col // 16) * 16
            lane = col - vbase
            iota = lax.iota(jnp.int32, 16)
            vec = buf[pl.ds(vbase, 16)]
            buf[pl.ds(vbase, 16)] = vec - jnp.where(iota == lane, _MS, 0.0)

        pltpu.sync_copy(buf, out_hbm.at[pl.ds(off, _CHUNK)])
        return carry

    lax.fori_loop(0, _ROWS_PER_W * _CHUNKS_PER_ROW, chunk_step, 0)


@jax.jit
def kernel(logits, labels):
    B, V = logits.shape
    flat = logits.reshape(B * V)
    labels32 = labels.astype(jnp.int32)
    mesh = plsc.VectorSubcoreMesh(core_axis_name="c", subcore_axis_name="s")
    run = pl.kernel(
        _sc_body,
        out_type=jax.ShapeDtypeStruct((B * V,), jnp.float32),
        mesh=mesh,
        scratch_types=[
            pltpu.VMEM((_CHUNK,), jnp.float32),
            pltpu.VMEM((_ROWS_PER_W + 16,), jnp.int32),
            pltpu.SemaphoreType.DMA,
        ],
    )
    return run(flat, labels32).reshape(B, V)
